# Initial kernel scaffold; baseline (speedup 1.0000x reference)
import jax
import jax.numpy as jnp
from jax.experimental import pallas as pl


def kernel(x, nbins):
    raise NotImplementedError("write your pallas kernel here")


# speedup vs baseline: 19.5030x; 19.5030x over previous
import functools

import jax
import jax.numpy as jnp
from jax import lax
from jax.experimental import pallas as pl
from jax.experimental.pallas import tpu as pltpu
from jax.experimental.pallas import tpu_sc as plsc

NB = 256
NC = 2
NS = 16
NW = NC * NS
L = 16


def _minmax_body(x_ref, min_ref, max_ref):
    first = (pl.program_id(0) == 0) & (pl.program_id(1) == 0)
    xb = x_ref[...]
    bmin = jnp.min(xb)
    bmax = jnp.max(xb)

    @pl.when(first)
    def _():
        min_ref[...] = jnp.reshape(bmin, (1, 1))
        max_ref[...] = jnp.reshape(bmax, (1, 1))

    @pl.when(jnp.logical_not(first))
    def _():
        min_ref[...] = jnp.minimum(min_ref[...], bmin)
        max_ref[...] = jnp.maximum(max_ref[...], bmax)


def _minmax(x2d):
    n, m = x2d.shape
    rpb, cpb = 8, m // 2
    return pl.pallas_call(
        _minmax_body,
        grid=(n // rpb, m // cpb),
        in_specs=[pl.BlockSpec((rpb, cpb), lambda i, j: (i, j))],
        out_specs=[
            pl.BlockSpec((1, 1), lambda i, j: (0, 0)),
            pl.BlockSpec((1, 1), lambda i, j: (0, 0)),
        ],
        out_shape=[
            jax.ShapeDtypeStruct((1, 1), jnp.float32),
            jax.ShapeDtypeStruct((1, 1), jnp.float32),
        ],
    )(x2d)


def _sc_hist(x2d, inv16, min16, elo, ehi):
    n, m = x2d.shape
    chunk = m // NW
    mesh = plsc.VectorSubcoreMesh(core_axis_name="c", subcore_axis_name="s")

    @functools.partial(
        pl.kernel,
        mesh=mesh,
        compiler_params=pltpu.CompilerParams(needs_layout_passes=False),
        out_type=jax.ShapeDtypeStruct((NW, n, NB), jnp.float32),
        scratch_types=[
            pltpu.VMEM((chunk,), jnp.float32),
            pltpu.VMEM((L * NB,), jnp.float32),
            pltpu.VMEM((n, NB), jnp.float32),
            pltpu.VMEM((L,), jnp.float32),
            pltpu.VMEM((L,), jnp.float32),
            pltpu.VMEM((NB,), jnp.float32),
            pltpu.VMEM((NB,), jnp.float32),
        ],
    )
    def hist_kernel(x_hbm, inv_hbm, min_hbm, elo_hbm, ehi_hbm, out_hbm,
                    xbuf, hist, outbuf, inv_v, minv_v, elo_v, ehi_v):
        c = lax.axis_index("c")
        s = lax.axis_index("s")
        wid = s * NC + c
        base = wid * chunk

        pltpu.sync_copy(inv_hbm, inv_v)
        pltpu.sync_copy(min_hbm, minv_v)
        pltpu.sync_copy(elo_hbm, elo_v)
        pltpu.sync_copy(ehi_hbm, ehi_v)
        inv = inv_v[...]
        minv = minv_v[...]
        lanebase = lax.iota(jnp.int32, L) * NB
        ones = jnp.ones((L,), jnp.float32)
        zeros = jnp.zeros((L,), jnp.float32)

        def _zero(j, _):
            hist[pl.ds(j * L, L)] = zeros
            return 0

        lax.fori_loop(0, (L * NB) // L, _zero, 0)

        def _row(r, _):
            pltpu.sync_copy(x_hbm.at[r, pl.ds(base, chunk)], xbuf)

            def _elems(i, _):
                v = xbuf[pl.ds(i * L, L)]
                k0 = ((v - minv) * inv).astype(jnp.int32)
                k0 = jnp.minimum(k0, NB - 1)
                e_lo = plsc.load_gather(elo_v, [k0])
                e_hi = plsc.load_gather(ehi_v, [k0])
                b = k0 + (v >= e_hi).astype(jnp.int32) - (v < e_lo).astype(jnp.int32)
                plsc.addupdate_scatter(hist, [b + lanebase], ones)
                return 0

            lax.fori_loop(0, chunk // L, _elems, 0)

            def _reduce(j, _):
                def _lane(l, acc):
                    off = l * NB + j * L
                    val = hist[pl.ds(off, L)]
                    hist[pl.ds(off, L)] = zeros
                    return acc + val

                acc = lax.fori_loop(0, L, _lane, zeros)
                outbuf[r, pl.ds(j * L, L)] = acc
                return 0

            lax.fori_loop(0, NB // L, _reduce, 0)
            return 0

        lax.fori_loop(0, n, _row, 0)
        pltpu.sync_copy(outbuf, out_hbm.at[wid])

    return hist_kernel(x2d, inv16, min16, elo, ehi)


def _csum(a):
    n, nb = a.shape
    c = a
    s = 1
    while s < nb:
        z = jnp.zeros((n, s), jnp.float32)
        c = c + jnp.concatenate([z, c[:, :-s]], axis=1)
        s *= 2
    return c


def _otsu_body(p_ref, min_ref, max_ref, thr_ref):
    n = thr_ref.shape[0]
    hist = jnp.sum(p_ref[...], axis=0)
    hs = jnp.sum(hist, axis=1, keepdims=True)
    hn = hist / hs
    binv = lax.broadcasted_iota(jnp.int32, (n, NB), 1).astype(jnp.float32)
    hb = hn * binv
    cw = _csum(hn)
    cs = _csum(hb)
    tw = jnp.sum(hn, axis=1, keepdims=True)
    ts = jnp.sum(hb, axis=1, keepdims=True)
    wbg = cw
    sbg = cs
    wfg = tw - wbg
    sfg = ts - sbg
    mbg = jnp.where(wbg > 0, sbg / jnp.where(wbg > 0, wbg, 1.0), 0.0)
    mfg = jnp.where(wfg > 0, sfg / jnp.where(wfg > 0, wfg, 1.0), 0.0)
    valid = (wbg > 0) & (wfg > 0)
    var = jnp.where(valid, wbg * wfg * (mbg - mfg) ** 2, -1.0)
    col = lax.broadcasted_iota(jnp.int32, (n, NB), 1)
    var = jnp.where(col == NB - 1, -1.0, var)
    mv = jnp.max(var, axis=1, keepdims=True)
    tmax = jnp.min(jnp.where(var == mv, col, NB), axis=1, keepdims=True)
    minv = min_ref[...]
    maxv = max_ref[...]
    edge = minv + (tmax.astype(jnp.float32) + 1.0) * (maxv - minv) / (NB - 1.0)
    best = jnp.where(mv > 0, edge, 0.0)
    thr_ref[...] = jnp.broadcast_to(best, (n, NB))


def _otsu(partials, min_v, max_v):
    nw, n, nb = partials.shape
    return pl.pallas_call(
        _otsu_body,
        in_specs=[
            pl.BlockSpec((nw, n, nb), lambda: (0, 0, 0)),
            pl.BlockSpec((1, 1), lambda: (0, 0)),
            pl.BlockSpec((1, 1), lambda: (0, 0)),
        ],
        out_specs=pl.BlockSpec((n, nb), lambda: (0, 0)),
        out_shape=jax.ShapeDtypeStruct((n, nb), jnp.float32),
    )(partials, min_v, max_v)


def _apply_body(x_ref, t_ref, o_ref):
    xb = x_ref[...]
    t = t_ref[:, 0:1]
    o_ref[...] = jnp.where(xb > t, xb, 0.0)


def _apply(x2d, thr):
    n, m = x2d.shape
    rpb, cpb = 8, m // 2
    return pl.pallas_call(
        _apply_body,
        grid=(n // rpb, m // cpb),
        in_specs=[
            pl.BlockSpec((rpb, cpb), lambda i, j: (i, j)),
            pl.BlockSpec((rpb, NB), lambda i, j: (i, 0)),
        ],
        out_specs=pl.BlockSpec((rpb, cpb), lambda i, j: (i, j)),
        out_shape=jax.ShapeDtypeStruct((n, m), jnp.float32),
    )(x2d, thr)


def kernel(x, nbins):
    b, c, h, w = x.shape
    n = b * c
    m = h * w
    x2d = x.reshape(n, m)
    min_v, max_v = _minmax(x2d)
    min_s = min_v[0, 0]
    width = max_v[0, 0] - min_s
    nf = jnp.asarray(nbins, jnp.float32)

    kf = jnp.arange(NB + 1, dtype=jnp.float32)
    edge = min_s + width * (kf / nf)

    def _pos(v):
        return (v - min_s) / width * nf

    for _ in range(12):
        edge = jnp.where(_pos(edge) >= kf, jnp.nextafter(edge, -jnp.inf), edge)
    for _ in range(12):
        edge = jnp.where(_pos(edge) < kf, jnp.nextafter(edge, jnp.inf), edge)
    elo = edge[:NB]
    ehi = jnp.concatenate([edge[1:NB], jnp.full((1,), jnp.inf, jnp.float32)])

    inv16 = jnp.broadcast_to(nf / width, (L,))
    min16 = jnp.broadcast_to(min_s, (L,))
    partials = _sc_hist(x2d, inv16, min16, elo, ehi)
    thr = _otsu(partials, min_v, max_v)
    out2d = _apply(x2d, thr)
    return out2d.reshape(x.shape), thr[:, 0]
